# Initial kernel scaffold; baseline (speedup 1.0000x reference)
import jax
import jax.numpy as jnp
from jax.experimental import pallas as pl


def kernel(inputs, weight, gates, k, sorted_expert_idxs, sorted_scattered_idxs, expert_offsets):
    raise NotImplementedError("write your pallas kernel here")


# speedup vs baseline: 1.4338x; 1.4338x over previous
import functools

import jax
import jax.numpy as jnp
from jax.experimental import pallas as pl
from jax.experimental.pallas import tpu as pltpu


def _grouped_matmul(x_rows, weight, g_row, starts, ends, *, br, bd):
    R, d_in = x_rows.shape
    E, d_out, _ = weight.shape
    nb = R // br
    njd = d_out // bd
    n_items = nb + E - 1

    fb = starts // br
    lb = jnp.where(ends > starts, (ends - 1) // br, fb - 1)
    nblk = jnp.maximum(lb - fb + 1, 0)
    csum = jnp.cumsum(nblk)
    item_base = csum - nblk
    total = csum[-1]
    ii = jnp.arange(n_items, dtype=jnp.int32)
    e_of_i = jnp.minimum(
        jnp.searchsorted(csum, ii, side="right").astype(jnp.int32), E - 1)
    blk = fb[e_of_i] + (ii - item_base[e_of_i])
    valid = ii < total
    block_id = jnp.where(valid, blk, nb - 1).astype(jnp.int32)
    expert_id = jnp.where(valid, e_of_i, E - 1).astype(jnp.int32)
    row_start = jnp.where(valid, jnp.maximum(starts[e_of_i], blk * br), 0)
    row_end = jnp.where(valid, jnp.minimum(ends[e_of_i], (blk + 1) * br), 0)
    first = jnp.concatenate([
        jnp.ones((1,), jnp.int32),
        (block_id[1:] != block_id[:-1]).astype(jnp.int32),
    ])

    def body(bid, eid, rs, re, ff, x_ref, w_ref, g_ref, y_ref):
        del eid
        i = pl.program_id(1)
        base = bid[i] * br
        rows = base + jax.lax.broadcasted_iota(jnp.int32, (br, 1), 0)
        sel = (rows >= rs[i]) & (rows < re[i])
        scale = jnp.where(sel, g_ref[...], 0.0)
        xs = x_ref[...] * scale
        acc = jax.lax.dot_general(
            xs, w_ref[0], (((1,), (1,)), ((), ())),
            preferred_element_type=jnp.float32)

        @pl.when(ff[i] == 1)
        def _():
            y_ref[...] = acc

        @pl.when(ff[i] == 0)
        def _():
            y_ref[...] += acc

    grid_spec = pltpu.PrefetchScalarGridSpec(
        num_scalar_prefetch=5,
        grid=(njd, n_items),
        in_specs=[
            pl.BlockSpec((br, d_in), lambda jd, i, bid, *_: (bid[i], 0)),
            pl.BlockSpec((1, bd, d_in),
                         lambda jd, i, bid, eid, *_: (eid[i], jd, 0)),
            pl.BlockSpec((br, 1), lambda jd, i, bid, *_: (bid[i], 0)),
        ],
        out_specs=pl.BlockSpec((br, bd), lambda jd, i, bid, *_: (bid[i], jd)),
    )
    return pl.pallas_call(
        body,
        grid_spec=grid_spec,
        out_shape=jax.ShapeDtypeStruct((R, d_out), jnp.float32),
        compiler_params=pltpu.CompilerParams(
            dimension_semantics=("arbitrary", "arbitrary")),
    )(block_id, expert_id, row_start, row_end, first, x_rows, weight, g_row)


def kernel(inputs, weight, gates, k, sorted_expert_idxs, sorted_scattered_idxs,
           expert_offsets):
    del k, sorted_expert_idxs
    n, kk = gates.shape
    ssi = sorted_scattered_idxs.astype(jnp.int32)
    offs = expert_offsets.astype(jnp.int32)
    E = weight.shape[0]
    starts = jnp.concatenate([jnp.zeros((1,), jnp.int32), offs[:-1]])
    ends = offs

    tok = ssi // kk
    x_rows = jnp.take(inputs, tok, axis=0)
    g_row = jnp.take(gates.reshape(-1), ssi, axis=0)[:, None]

    y = _grouped_matmul(x_rows, weight, g_row, starts, ends, br=256, bd=512)

    inv = jnp.argsort(ssi)
    out = jnp.take(y, inv, axis=0)
    return out.reshape(n, kk, -1).sum(axis=1)


# speedup vs baseline: 2.3430x; 1.6341x over previous
import functools

import jax
import jax.numpy as jnp
from jax import lax
from jax.experimental import pallas as pl
from jax.experimental.pallas import tpu as pltpu
from jax.experimental.pallas import tpu_sc as plsc

_NC = 2
_NS = 16
_NW = _NC * _NS
_LANES = 16


def _sc_dispatch_gather(inputs, tok):
    R = tok.shape[0]
    _, d_in = inputs.shape
    rpw = R // _NW
    chunk = 64
    n_chunks = rpw // chunk
    mesh = plsc.VectorSubcoreMesh(core_axis_name="c", subcore_axis_name="s")

    @functools.partial(
        pl.kernel,
        out_type=jax.ShapeDtypeStruct((R, d_in), jnp.float32),
        mesh=mesh,
        scratch_types=(
            pltpu.VMEM((chunk,), jnp.int32),
            pltpu.VMEM((chunk, d_in), jnp.float32),
            pltpu.SemaphoreType.DMA,
        ),
    )
    def run(inputs_hbm, tok_hbm, xrows_hbm, idx_v, rows_v, sem):
        wid = lax.axis_index("s") * _NC + lax.axis_index("c")
        base = wid * rpw
        for c in range(n_chunks):
            cb = base + c * chunk
            pltpu.sync_copy(tok_hbm.at[pl.ds(cb, chunk)], idx_v)
            pltpu.async_copy(inputs_hbm.at[idx_v], rows_v, sem).wait()
            pltpu.sync_copy(rows_v, xrows_hbm.at[pl.ds(cb, chunk)])

    return run(inputs, tok)


def _sc_combine(y, inv, gates_flat, n_tokens, kk):
    R, d_out = y.shape
    tpw = n_tokens // _NW
    ct = 16
    n_chunks = tpw // ct
    vregs = d_out // _LANES
    mesh = plsc.VectorSubcoreMesh(core_axis_name="c", subcore_axis_name="s")

    @functools.partial(
        pl.kernel,
        out_type=jax.ShapeDtypeStruct((n_tokens, d_out), jnp.float32),
        mesh=mesh,
        scratch_types=(
            pltpu.VMEM((ct * kk,), jnp.int32),
            pltpu.VMEM((ct * kk, d_out), jnp.float32),
            pltpu.VMEM((ct, d_out), jnp.float32),
            pltpu.VMEM((ct * kk,), jnp.float32),
            pltpu.SemaphoreType.DMA,
        ),
    )
    def run(y_hbm, inv_hbm, gates_hbm, res_hbm, idx_v, ybuf_v, obuf_v, g_v,
            sem):
        wid = lax.axis_index("s") * _NC + lax.axis_index("c")
        tbase = wid * tpw
        for c in range(n_chunks):
            tb = tbase + c * ct
            pltpu.sync_copy(inv_hbm.at[pl.ds(tb * kk, ct * kk)], idx_v)
            pltpu.sync_copy(gates_hbm.at[pl.ds(tb * kk, ct * kk)], g_v)
            pltpu.async_copy(y_hbm.at[idx_v], ybuf_v, sem).wait()
            gregs = [g_v[pl.ds(m * _LANES, _LANES)]
                     for m in range(ct * kk // _LANES)]
            gs = [gregs[i // _LANES][i % _LANES] for i in range(ct * kk)]

            def body(j, _):
                for t in range(ct):
                    acc = gs[t * kk] * ybuf_v[t * kk,
                                              pl.ds(j * _LANES, _LANES)]
                    for jj in range(1, kk):
                        acc = acc + gs[t * kk + jj] * ybuf_v[
                            t * kk + jj, pl.ds(j * _LANES, _LANES)]
                    obuf_v[t, pl.ds(j * _LANES, _LANES)] = acc
                return 0

            lax.fori_loop(0, vregs, body, 0)
            pltpu.sync_copy(obuf_v, res_hbm.at[pl.ds(tb, ct)])

    return run(y, inv, gates_flat)


def _grouped_matmul(x_rows, weight, starts, ends, *, br, bd):
    R, d_in = x_rows.shape
    E, d_out, _ = weight.shape
    nb = R // br
    njd = d_out // bd
    n_items = nb + E - 1

    fb = starts // br
    lb = jnp.where(ends > starts, (ends - 1) // br, fb - 1)
    nblk = jnp.maximum(lb - fb + 1, 0)
    csum = jnp.cumsum(nblk)
    item_base = csum - nblk
    total = csum[-1]
    ii = jnp.arange(n_items, dtype=jnp.int32)
    e_of_i = jnp.minimum(
        jnp.searchsorted(csum, ii, side="right").astype(jnp.int32), E - 1)
    blk = fb[e_of_i] + (ii - item_base[e_of_i])
    valid = ii < total
    block_id = jnp.where(valid, blk, nb - 1).astype(jnp.int32)
    expert_id = jnp.where(valid, e_of_i, E - 1).astype(jnp.int32)
    row_start = jnp.where(valid, jnp.maximum(starts[e_of_i], blk * br), 0)
    row_end = jnp.where(valid, jnp.minimum(ends[e_of_i], (blk + 1) * br), 0)
    first = jnp.concatenate([
        jnp.ones((1,), jnp.int32),
        (block_id[1:] != block_id[:-1]).astype(jnp.int32),
    ])

    def body(bid, eid, rs, re, ff, x_ref, w_ref, y_ref):
        del eid
        i = pl.program_id(1)
        base = bid[i] * br
        rows = base + jax.lax.broadcasted_iota(jnp.int32, (br, 1), 0)
        sel = (rows >= rs[i]) & (rows < re[i])
        xs = jnp.where(sel, x_ref[...], 0.0)
        acc = jax.lax.dot_general(
            xs, w_ref[0], (((1,), (1,)), ((), ())),
            preferred_element_type=jnp.float32)

        @pl.when(ff[i] == 1)
        def _():
            y_ref[...] = acc

        @pl.when(ff[i] == 0)
        def _():
            y_ref[...] += acc

    grid_spec = pltpu.PrefetchScalarGridSpec(
        num_scalar_prefetch=5,
        grid=(njd, n_items),
        in_specs=[
            pl.BlockSpec((br, d_in), lambda jd, i, bid, *_: (bid[i], 0)),
            pl.BlockSpec((1, bd, d_in),
                         lambda jd, i, bid, eid, *_: (eid[i], jd, 0)),
        ],
        out_specs=pl.BlockSpec((br, bd), lambda jd, i, bid, *_: (bid[i], jd)),
    )
    return pl.pallas_call(
        body,
        grid_spec=grid_spec,
        out_shape=jax.ShapeDtypeStruct((R, d_out), jnp.float32),
        compiler_params=pltpu.CompilerParams(
            dimension_semantics=("arbitrary", "arbitrary")),
    )(block_id, expert_id, row_start, row_end, first, x_rows, weight)


def kernel(inputs, weight, gates, k, sorted_expert_idxs, sorted_scattered_idxs,
           expert_offsets):
    del k, sorted_expert_idxs
    n, kk = gates.shape
    ssi = sorted_scattered_idxs.astype(jnp.int32)
    offs = expert_offsets.astype(jnp.int32)
    starts = jnp.concatenate([jnp.zeros((1,), jnp.int32), offs[:-1]])
    ends = offs

    tok = ssi // kk
    x_rows = _sc_dispatch_gather(inputs, tok)
    y = _grouped_matmul(x_rows, weight, starts, ends, br=256, bd=512)
    inv = jnp.zeros((n * kk,), jnp.int32).at[ssi].set(
        jnp.arange(n * kk, dtype=jnp.int32))
    return _sc_combine(y, inv, gates.reshape(-1), n, kk)


# speedup vs baseline: 2.5234x; 1.0770x over previous
import functools

import jax
import jax.numpy as jnp
from jax import lax
from jax.experimental import pallas as pl
from jax.experimental.pallas import tpu as pltpu
from jax.experimental.pallas import tpu_sc as plsc

_NC = 2
_NS = 16
_NW = _NC * _NS
_LANES = 16


def _sc_dispatch_gather(inputs, tok):
    R = tok.shape[0]
    _, d_in = inputs.shape
    rpw = R // _NW
    chunk = 64
    n_chunks = rpw // chunk
    mesh = plsc.VectorSubcoreMesh(core_axis_name="c", subcore_axis_name="s")

    @functools.partial(
        pl.kernel,
        out_type=jax.ShapeDtypeStruct((R, d_in), jnp.float32),
        mesh=mesh,
        scratch_types=(
            pltpu.VMEM((chunk,), jnp.int32),
            pltpu.VMEM((chunk, d_in), jnp.float32),
            pltpu.SemaphoreType.DMA,
        ),
    )
    def run(inputs_hbm, tok_hbm, xrows_hbm, idx_v, rows_v, sem):
        wid = lax.axis_index("s") * _NC + lax.axis_index("c")
        base = wid * rpw
        for c in range(n_chunks):
            cb = base + c * chunk
            pltpu.sync_copy(tok_hbm.at[pl.ds(cb, chunk)], idx_v)
            pltpu.async_copy(inputs_hbm.at[idx_v], rows_v, sem).wait()
            pltpu.sync_copy(rows_v, xrows_hbm.at[pl.ds(cb, chunk)])

    return run(inputs, tok)


def _sc_combine(y, inv, gates_flat, n_tokens, kk):
    R, d_out = y.shape
    tpw = n_tokens // _NW
    ct = _LANES // kk
    n_chunks = tpw // ct
    vregs = d_out // _LANES
    mesh = plsc.VectorSubcoreMesh(core_axis_name="c", subcore_axis_name="s")

    @functools.partial(
        pl.kernel,
        out_type=jax.ShapeDtypeStruct((n_tokens, d_out), jnp.float32),
        mesh=mesh,
        scratch_types=(
            pltpu.VMEM((tpw * kk,), jnp.int32),
            pltpu.VMEM((tpw * kk,), jnp.float32),
            pltpu.VMEM((2, ct * kk, d_out), jnp.float32),
            pltpu.VMEM((2, ct, d_out), jnp.float32),
            pltpu.SemaphoreType.DMA,
            pltpu.SemaphoreType.DMA,
            pltpu.SemaphoreType.DMA,
            pltpu.SemaphoreType.DMA,
        ),
    )
    def run(y_hbm, inv_hbm, gates_hbm, res_hbm, idx_v, g_v, ybuf_v, obuf_v,
            gsem0, gsem1, ssem0, ssem1):
        wid = lax.axis_index("s") * _NC + lax.axis_index("c")
        tbase = wid * tpw
        pltpu.sync_copy(inv_hbm.at[pl.ds(tbase * kk, tpw * kk)], idx_v)
        pltpu.sync_copy(gates_hbm.at[pl.ds(tbase * kk, tpw * kk)], g_v)
        gsems = (gsem0, gsem1)
        ssems = (ssem0, ssem1)

        def start_gather(c):
            b = c % 2
            return pltpu.async_copy(
                y_hbm.at[idx_v.at[pl.ds(c * ct * kk, ct * kk)]],
                ybuf_v.at[b], gsems[b])

        gh = {0: start_gather(0)}
        sh = {}
        for c in range(n_chunks):
            b = c % 2
            if c + 1 < n_chunks:
                gh[c + 1] = start_gather(c + 1)
            gh.pop(c).wait()
            if c >= 2:
                sh.pop(c - 2).wait()
            greg = g_v[pl.ds(c * ct * kk, _LANES)]
            gs = [greg[i] for i in range(ct * kk)]

            def body(j, _):
                for t in range(ct):
                    acc = gs[t * kk] * ybuf_v[b, t * kk,
                                              pl.ds(j * _LANES, _LANES)]
                    for jj in range(1, kk):
                        acc = acc + gs[t * kk + jj] * ybuf_v[
                            b, t * kk + jj, pl.ds(j * _LANES, _LANES)]
                    obuf_v[b, t, pl.ds(j * _LANES, _LANES)] = acc
                return 0

            lax.fori_loop(0, vregs, body, 0)
            sh[c] = pltpu.async_copy(
                obuf_v.at[b], res_hbm.at[pl.ds(tbase + c * ct, ct)], ssems[b])
        for c in sorted(sh):
            sh.pop(c).wait()

    return run(y, inv, gates_flat)


def _grouped_matmul(x_rows, weight, starts, ends, *, br, bd):
    R, d_in = x_rows.shape
    E, d_out, _ = weight.shape
    nb = R // br
    njd = d_out // bd
    n_items = nb + E - 1

    fb = starts // br
    lb = jnp.where(ends > starts, (ends - 1) // br, fb - 1)
    nblk = jnp.maximum(lb - fb + 1, 0)
    csum = jnp.cumsum(nblk)
    item_base = csum - nblk
    total = csum[-1]
    ii = jnp.arange(n_items, dtype=jnp.int32)
    e_of_i = jnp.minimum(
        jnp.searchsorted(csum, ii, side="right").astype(jnp.int32), E - 1)
    blk = fb[e_of_i] + (ii - item_base[e_of_i])
    valid = ii < total
    block_id = jnp.where(valid, blk, nb - 1).astype(jnp.int32)
    expert_id = jnp.where(valid, e_of_i, E - 1).astype(jnp.int32)
    row_start = jnp.where(valid, jnp.maximum(starts[e_of_i], blk * br), 0)
    row_end = jnp.where(valid, jnp.minimum(ends[e_of_i], (blk + 1) * br), 0)
    first = jnp.concatenate([
        jnp.ones((1,), jnp.int32),
        (block_id[1:] != block_id[:-1]).astype(jnp.int32),
    ])

    def body(bid, eid, rs, re, ff, x_ref, w_ref, y_ref):
        del eid
        i = pl.program_id(1)
        base = bid[i] * br
        rows = base + jax.lax.broadcasted_iota(jnp.int32, (br, 1), 0)
        sel = (rows >= rs[i]) & (rows < re[i])
        xs = jnp.where(sel, x_ref[...], 0.0)
        acc = jax.lax.dot_general(
            xs, w_ref[0], (((1,), (1,)), ((), ())),
            preferred_element_type=jnp.float32)

        @pl.when(ff[i] == 1)
        def _():
            y_ref[...] = acc

        @pl.when(ff[i] == 0)
        def _():
            y_ref[...] += acc

    grid_spec = pltpu.PrefetchScalarGridSpec(
        num_scalar_prefetch=5,
        grid=(njd, n_items),
        in_specs=[
            pl.BlockSpec((br, d_in), lambda jd, i, bid, *_: (bid[i], 0)),
            pl.BlockSpec((1, bd, d_in),
                         lambda jd, i, bid, eid, *_: (eid[i], jd, 0)),
        ],
        out_specs=pl.BlockSpec((br, bd), lambda jd, i, bid, *_: (bid[i], jd)),
    )
    return pl.pallas_call(
        body,
        grid_spec=grid_spec,
        out_shape=jax.ShapeDtypeStruct((R, d_out), jnp.float32),
        compiler_params=pltpu.CompilerParams(
            dimension_semantics=("arbitrary", "arbitrary")),
    )(block_id, expert_id, row_start, row_end, first, x_rows, weight)


def kernel(inputs, weight, gates, k, sorted_expert_idxs, sorted_scattered_idxs,
           expert_offsets):
    del k, sorted_expert_idxs
    n, kk = gates.shape
    ssi = sorted_scattered_idxs.astype(jnp.int32)
    offs = expert_offsets.astype(jnp.int32)
    starts = jnp.concatenate([jnp.zeros((1,), jnp.int32), offs[:-1]])
    ends = offs

    tok = ssi // kk
    x_rows = _sc_dispatch_gather(inputs, tok)
    y = _grouped_matmul(x_rows, weight, starts, ends, br=256, bd=512)
    inv = jnp.zeros((n * kk,), jnp.int32).at[ssi].set(
        jnp.arange(n * kk, dtype=jnp.int32))
    return _sc_combine(y, inv, gates.reshape(-1), n, kk)
